# baseline (speedup 1.0000x reference)
import jax
import jax.numpy as jnp
from jax.experimental import pallas as pl
from jax.experimental.pallas import tpu as pltpu

_B, _N, _M = 16, 2048, 2048
_GAMMA = 0.5
_RB = 512
_NBLK = _N // _RB
_NEG = float("-inf")


def _triplet_body(scores_ref, g0_ref, g1_ref, lr_ref, out_ref,
                  cv1_ref, cv2_ref, cpos_ref, acc_ref):
    b = pl.program_id(0)
    rb = pl.program_id(1)
    s = scores_ref[0]
    g0 = g0_ref[0, 0]
    g1 = g1_ref[0]
    lr = lr_ref[0]

    @pl.when(jnp.logical_and(b == 0, rb == 0))
    def _():
        acc_ref[0, 0] = 0.0

    @pl.when(rb == 0)
    def _():
        cv1_ref[...] = jnp.full((1, _M), _NEG, jnp.float32)
        cv2_ref[...] = jnp.full((1, _M), _NEG, jnp.float32)
        cpos_ref[...] = jnp.zeros((1, _M), jnp.float32)

    col_idx = jax.lax.broadcasted_iota(jnp.int32, (_RB, _M + 1), 1)
    rm1 = jnp.max(s, axis=1, keepdims=True)
    rm2 = jnp.max(jnp.where(s == rm1, _NEG, s), axis=1, keepdims=True)
    rpos = jnp.sum(jnp.where(col_idx == g0, s, 0.0), axis=1, keepdims=True)
    neg = jnp.where(rpos == rm1, rm2, rm1)
    acc_ref[0, 0] += jnp.sum(jnp.maximum(neg - rpos + _GAMMA, 0.0))

    sc = s[:, :_M]
    row_idx = jax.lax.broadcasted_iota(jnp.int32, (_RB, _M), 0)
    cm1 = jnp.max(sc, axis=0, keepdims=True)
    cm2 = jnp.max(jnp.where(sc == cm1, _NEG, sc), axis=0, keepdims=True)
    cpos_ref[...] += jnp.sum(jnp.where(row_idx == (g1 - rb * _RB), sc, 0.0),
                             axis=0, keepdims=True)
    pv1, pv2 = cv1_ref[...], cv2_ref[...]
    cv1_ref[...] = jnp.maximum(pv1, cm1)
    cv2_ref[...] = jnp.maximum(jnp.maximum(pv2, cm2), jnp.minimum(pv1, cm1))

    @pl.when(rb == _NBLK - 1)
    def _():
        pv1, pv2 = cv1_ref[...], cv2_ref[...]
        fv1 = jnp.maximum(pv1, lr)
        fv2 = jnp.maximum(jnp.minimum(pv1, lr), pv2)
        fpos = jnp.where(g1 == _N, lr, cpos_ref[...])
        fneg = jnp.where(fpos == fv1, fv2, fv1)
        acc_ref[0, 0] += jnp.sum(jnp.maximum(fneg - fpos + _GAMMA, 0.0))

    out_ref[...] = jnp.full((1, 1), acc_ref[0, 0] * (1.0 / (2 * _B * _N)),
                            jnp.float32)


def _run(scores, g0r, g1r, lr):
    return pl.pallas_call(
        _triplet_body,
        grid=(_B, _NBLK),
        in_specs=[
            pl.BlockSpec((1, _RB, _M + 1), lambda b, rb: (b, rb, 0)),
            pl.BlockSpec((1, 1, _RB, 1), lambda b, rb: (b, rb, 0, 0)),
            pl.BlockSpec((1, 1, _M), lambda b, rb: (b, 0, 0)),
            pl.BlockSpec((1, 1, _M), lambda b, rb: (b, 0, 0)),
        ],
        out_specs=pl.BlockSpec((1, 1), lambda b, rb: (0, 0)),
        out_shape=jax.ShapeDtypeStruct((1, 1), jnp.float32),
        scratch_shapes=[
            pltpu.VMEM((1, _M), jnp.float32),
            pltpu.VMEM((1, _M), jnp.float32),
            pltpu.VMEM((1, _M), jnp.float32),
            pltpu.SMEM((1, 1), jnp.float32),
        ],
    )(scores, g0r, g1r, lr)


def kernel(gt_matches0, gt_matches1, scores):
    g0 = jnp.where(gt_matches0 == -1, _M, gt_matches0).astype(jnp.int32)
    g1 = jnp.where(gt_matches1 == -1, _N, gt_matches1).astype(jnp.int32)
    g0r = g0.reshape(_B, _NBLK, _RB, 1)
    g1r = g1.reshape(_B, 1, _M)
    lr = scores[:, _N, :_M].reshape(_B, 1, _M)
    out = _run(scores, g0r, g1r, lr)
    return out[0, 0]


# speedup vs baseline: 1.0975x; 1.0975x over previous
import jax
import jax.numpy as jnp
from jax.experimental import pallas as pl
from jax.experimental.pallas import tpu as pltpu

_B, _N, _M = 16, 2048, 2048
_GAMMA = 0.5
_RB = 512
_NCH = _N // _RB
_NEG = float("-inf")


def _triplet_body(scores_hbm, g0_ref, g1_ref, lr_ref, out_ref,
                  buf_ref, acc_ref, sems):
    b = pl.program_id(0)
    g0 = g0_ref[0, 0]
    g1 = g1_ref[0]
    lr = lr_ref[0]

    def start_copy(bb, rb):
        pltpu.make_async_copy(
            scores_hbm.at[bb, pl.ds(rb * _RB, _RB), :],
            buf_ref.at[rb],
            sems.at[rb],
        ).start()

    @pl.when(b == 0)
    def _():
        acc_ref[0, 0] = 0.0
        for rb in range(_NCH):
            start_copy(0, rb)

    cv1 = jnp.full((1, _M), _NEG, jnp.float32)
    cv2 = jnp.full((1, _M), _NEG, jnp.float32)
    cpos = jnp.zeros((1, _M), jnp.float32)
    part = jnp.zeros((1, 1), jnp.float32)

    for rb in range(_NCH):
        pltpu.make_async_copy(
            scores_hbm.at[b, pl.ds(rb * _RB, _RB), :],
            buf_ref.at[rb],
            sems.at[rb],
        ).wait()
        s = buf_ref[rb]

        col_idx = jax.lax.broadcasted_iota(jnp.int32, (_RB, _M + 1), 1)
        g0b = g0[rb * _RB:(rb + 1) * _RB, :]
        rm1 = jnp.max(s, axis=1, keepdims=True)
        rm2 = jnp.max(jnp.where(s == rm1, _NEG, s), axis=1, keepdims=True)
        rpos = jnp.sum(jnp.where(col_idx == g0b, s, 0.0), axis=1,
                       keepdims=True)
        neg = jnp.where(rpos == rm1, rm2, rm1)
        part += jnp.sum(jnp.maximum(neg - rpos + _GAMMA, 0.0)).reshape(1, 1)

        sc = s[:, :_M]
        row_idx = jax.lax.broadcasted_iota(jnp.int32, (_RB, _M), 0)
        cm1 = jnp.max(sc, axis=0, keepdims=True)
        cm2 = jnp.max(jnp.where(sc == cm1, _NEG, sc), axis=0, keepdims=True)
        cpos = cpos + jnp.sum(
            jnp.where(row_idx == (g1 - rb * _RB), sc, 0.0),
            axis=0, keepdims=True)
        cv2 = jnp.maximum(jnp.maximum(cv2, cm2), jnp.minimum(cv1, cm1))
        cv1 = jnp.maximum(cv1, cm1)

        @pl.when(b < _B - 1)
        def _():
            start_copy(b + 1, rb)

    fv1 = jnp.maximum(cv1, lr)
    fv2 = jnp.maximum(jnp.minimum(cv1, lr), cv2)
    fpos = jnp.where(g1 == _N, lr, cpos)
    fneg = jnp.where(fpos == fv1, fv2, fv1)
    part += jnp.sum(jnp.maximum(fneg - fpos + _GAMMA, 0.0)).reshape(1, 1)

    acc_ref[0, 0] += part[0, 0]
    out_ref[...] = jnp.full((1, 1), acc_ref[0, 0] * (1.0 / (2 * _B * _N)),
                            jnp.float32)


def _run(scores, g0r, g1r, lr):
    return pl.pallas_call(
        _triplet_body,
        grid=(_B,),
        in_specs=[
            pl.BlockSpec(memory_space=pl.ANY),
            pl.BlockSpec((1, 1, _N, 1), lambda b: (b, 0, 0, 0)),
            pl.BlockSpec((1, 1, _M), lambda b: (b, 0, 0)),
            pl.BlockSpec((1, 1, _M), lambda b: (b, 0, 0)),
        ],
        out_specs=pl.BlockSpec((1, 1), lambda b: (0, 0)),
        out_shape=jax.ShapeDtypeStruct((1, 1), jnp.float32),
        scratch_shapes=[
            pltpu.VMEM((_NCH, _RB, _M + 1), jnp.float32),
            pltpu.SMEM((1, 1), jnp.float32),
            pltpu.SemaphoreType.DMA((_NCH,)),
        ],
    )(scores, g0r, g1r, lr)


def kernel(gt_matches0, gt_matches1, scores):
    g0 = jnp.where(gt_matches0 == -1, _M, gt_matches0).astype(jnp.int32)
    g1 = jnp.where(gt_matches1 == -1, _N, gt_matches1).astype(jnp.int32)
    g0r = g0.reshape(_B, 1, _N, 1)
    g1r = g1.reshape(_B, 1, _M)
    lr = scores[:, _N, :_M].reshape(_B, 1, _M)
    out = _run(scores, g0r, g1r, lr)
    return out[0, 0]
